# Initial kernel scaffold; baseline (speedup 1.0000x reference)
import jax
import jax.numpy as jnp
from jax.experimental import pallas as pl


def kernel(x, batch, vn_embedding, ln_g, ln_b, W1, b1, W2, b2):
    raise NotImplementedError("write your pallas kernel here")


# speedup vs baseline: 1.8366x; 1.8366x over previous
import dataclasses

import jax
import jax.numpy as jnp
from jax import lax
from jax.experimental import pallas as pl
from jax.experimental.pallas import tpu as pltpu
from jax.experimental.pallas import tpu_sc as plsc

N = 320000
H = 128
G = 512
NC = 2
NS = 16
NW = NC * NS
ROWS_W = N // NW
CH = 80
NCHUNK = ROWS_W // CH
EPS = 1e-5
_SQRT2 = 1.4142135623730951


def _row0(wid, c):
    return wid * ROWS_W + c * CH


def _seg_sum_body(x_hbm, b3_hbm, sums_hbm, cnts_hbm,
                  xb0, xb1, xb2, xb3, idx_all, cnt_local, acc,
                  sx0, sx1, sx2, sx3, sc0, sc1, sc2, sc3):
    cid = lax.axis_index("core")
    sid = lax.axis_index("subcore")
    wid = cid * NS + sid
    xb = (xb0, xb1, xb2, xb3)
    sx = (sx0, sx1, sx2, sx3)
    ssc = (sc0, sc1, sc2, sc3)

    z16 = jnp.zeros((16,), jnp.float32)

    @pl.loop(0, 32)
    def _(i):
        for k in range(H // 16):
            xb0[i, pl.ds(k * 16, 16)] = z16

    @pl.loop(0, G // 16)
    def _(i):
        cnt_local[pl.ds(i * 16, 16)] = z16

    pltpu.sync_copy(xb0.at[pl.ds(0, 32)], acc.at[pl.ds(sid * 32, 32)])
    pltpu.sync_copy(b3_hbm.at[wid], idx_all)

    for b in range(2):
        pltpu.async_copy(x_hbm.at[pl.ds(_row0(wid, b), CH)], xb[b], sx[b])

    plsc.subcore_barrier()
    o16 = jnp.ones((16,), jnp.float32)

    def chunk_step(c, b, bi, issue_next):
        pltpu.make_async_copy(x_hbm.at[pl.ds(0, CH)], xb[b], sx[b]).wait()
        for j in range(CH // 16):
            idv = idx_all[c, pl.ds(j * 16, 16)]
            plsc.addupdate_scatter(cnt_local, [idv], o16)
        pltpu.async_copy(xb[b], acc.at[idx_all.at[c]], ssc[b], add=True)
        if issue_next:
            @pl.when(c + 2 < NCHUNK)
            def _():
                @pl.when(c >= 2)
                def _():
                    pltpu.make_async_copy(xb[bi], acc.at[idx_all.at[0]],
                                          ssc[bi]).wait()
                pltpu.async_copy(x_hbm.at[pl.ds(_row0(wid, c) + 2 * CH, CH)],
                                 xb[bi], sx[bi])

    @pl.loop(0, NCHUNK - 1, step=4)
    def _(cc):
        for b in range(4):
            chunk_step(cc + b, b, (b + 2) % 4, True)

    chunk_step(NCHUNK - 1, (NCHUNK - 1) % 4, (NCHUNK + 1) % 4, False)

    for b in range(4):
        pltpu.make_async_copy(xb[b], acc.at[idx_all.at[0]], ssc[b]).wait()

    pltpu.sync_copy(cnt_local, cnts_hbm.at[wid])
    plsc.subcore_barrier()
    pltpu.sync_copy(acc.at[pl.ds(sid * 32, 32)],
                    sums_hbm.at[cid, pl.ds(sid * 32, 32)])


def _mlp_body(sums_ref, cnts_ref, emb_ref, g_ref, b_ref, w1_ref, b1_ref,
              w2_ref, b2_ref, h_ref):
    sums = sums_ref[0] + sums_ref[1]
    cnt = jnp.sum(cnts_ref[...], axis=0)
    gx = sums / jnp.maximum(cnt, 1.0)[:, None]
    vn = emb_ref[...] + gx
    mu = jnp.mean(vn, axis=-1, keepdims=True)
    var = jnp.mean((vn - mu) * (vn - mu), axis=-1, keepdims=True)
    hn = (vn - mu) * lax.rsqrt(var + EPS) * g_ref[...] + b_ref[...]
    h1 = jnp.dot(hn, w1_ref[...], preferred_element_type=jnp.float32)
    h1 = h1 + b1_ref[...]
    h1 = 0.5 * h1 * (1.0 + lax.erf(h1 / _SQRT2))
    h2 = jnp.dot(h1, w2_ref[...], preferred_element_type=jnp.float32)
    h_ref[...] = h2 + b2_ref[...]


def _apply_body(x_hbm, b3_hbm, h_hbm, out_hbm,
                xb0, xb1, hb0, hb1, ob0, ob1, idx_all,
                sx0, sx1, sg0, sg1, so0, so1):
    cid = lax.axis_index("core")
    sid = lax.axis_index("subcore")
    wid = cid * NS + sid
    xb = (xb0, xb1)
    hb = (hb0, hb1)
    ob = (ob0, ob1)
    sx = (sx0, sx1)
    sg = (sg0, sg1)
    so = (so0, so1)

    pltpu.sync_copy(b3_hbm.at[wid], idx_all)
    for b in range(2):
        pltpu.async_copy(x_hbm.at[pl.ds(_row0(wid, b), CH)], xb[b], sx[b])
        pltpu.async_copy(h_hbm.at[idx_all.at[b]], hb[b], sg[b])

    def chunk_step(c, b, issue_next):
        pltpu.make_async_copy(x_hbm.at[pl.ds(0, CH)], xb[b], sx[b]).wait()
        pltpu.make_async_copy(h_hbm.at[idx_all.at[0]], hb[b], sg[b]).wait()

        @pl.when(c >= 2)
        def _():
            pltpu.make_async_copy(ob[b], out_hbm.at[pl.ds(0, CH)],
                                  so[b]).wait()

        @pl.loop(0, CH)
        def _(i):
            for k in range(H // 16):
                s = pl.ds(k * 16, 16)
                ob[b][i, s] = xb[b][i, s] + hb[b][i, s]

        pltpu.async_copy(ob[b], out_hbm.at[pl.ds(_row0(wid, c), CH)], so[b])
        if issue_next:
            @pl.when(c + 2 < NCHUNK)
            def _():
                pltpu.async_copy(
                    x_hbm.at[pl.ds(_row0(wid, c) + 2 * CH, CH)], xb[b], sx[b])
                pltpu.async_copy(h_hbm.at[idx_all.at[c + 2]], hb[b], sg[b])

    @pl.loop(0, NCHUNK - 1, step=2)
    def _(cc):
        for b in range(2):
            chunk_step(cc + b, b, True)

    chunk_step(NCHUNK - 1, (NCHUNK - 1) % 2, False)
    for b in range(2):
        pltpu.make_async_copy(ob[b], out_hbm.at[pl.ds(0, CH)], so[b]).wait()


_sc_mesh = plsc.VectorSubcoreMesh(core_axis_name="core",
                                  subcore_axis_name="subcore")

_sc_cp = pltpu.CompilerParams()
if "needs_layout_passes" in pltpu.CompilerParams.__dataclass_fields__:
    _sc_cp = dataclasses.replace(_sc_cp, needs_layout_passes=False)

_seg_sum = pl.kernel(
    _seg_sum_body,
    out_type=(jax.ShapeDtypeStruct((NC, G, H), jnp.float32),
              jax.ShapeDtypeStruct((NW, G), jnp.float32)),
    mesh=_sc_mesh,
    compiler_params=_sc_cp,
    name="seg_sum_sc",
    scratch_types=[
        pltpu.VMEM((CH, H), jnp.float32),
        pltpu.VMEM((CH, H), jnp.float32),
        pltpu.VMEM((CH, H), jnp.float32),
        pltpu.VMEM((CH, H), jnp.float32),
        pltpu.VMEM((NCHUNK, CH), jnp.int32),
        pltpu.VMEM((G,), jnp.float32),
        pltpu.VMEM_SHARED((G, H), jnp.float32),
        pltpu.SemaphoreType.DMA,
        pltpu.SemaphoreType.DMA,
        pltpu.SemaphoreType.DMA,
        pltpu.SemaphoreType.DMA,
        pltpu.SemaphoreType.DMA,
        pltpu.SemaphoreType.DMA,
        pltpu.SemaphoreType.DMA,
        pltpu.SemaphoreType.DMA,
    ],
)

_apply = pl.kernel(
    _apply_body,
    out_type=jax.ShapeDtypeStruct((N, H), jnp.float32),
    mesh=_sc_mesh,
    compiler_params=_sc_cp,
    name="apply_sc",
    scratch_types=[
        pltpu.VMEM((CH, H), jnp.float32),
        pltpu.VMEM((CH, H), jnp.float32),
        pltpu.VMEM((CH, H), jnp.float32),
        pltpu.VMEM((CH, H), jnp.float32),
        pltpu.VMEM((CH, H), jnp.float32),
        pltpu.VMEM((CH, H), jnp.float32),
        pltpu.VMEM((NCHUNK, CH), jnp.int32),
        pltpu.SemaphoreType.DMA,
        pltpu.SemaphoreType.DMA,
        pltpu.SemaphoreType.DMA,
        pltpu.SemaphoreType.DMA,
        pltpu.SemaphoreType.DMA,
        pltpu.SemaphoreType.DMA,
    ],
)

_mlp = pl.pallas_call(
    _mlp_body,
    out_shape=jax.ShapeDtypeStruct((G, H), jnp.float32),
)


def kernel(x, batch, vn_embedding, ln_g, ln_b, W1, b1, W2, b2):
    batch3 = batch.astype(jnp.int32).reshape(NW, NCHUNK, CH)
    sums_p, cnts_p = _seg_sum(x, batch3)
    h = _mlp(sums_p, cnts_p, vn_embedding,
             ln_g.reshape(1, H), ln_b.reshape(1, H),
             W1, b1.reshape(1, H), W2, b2.reshape(1, H))
    x_out = _apply(x, batch3, h)
    return (x_out, h)


# speedup vs baseline: 1.8778x; 1.0224x over previous
import dataclasses

import jax
import jax.numpy as jnp
from jax import lax
from jax.experimental import pallas as pl
from jax.experimental.pallas import tpu as pltpu
from jax.experimental.pallas import tpu_sc as plsc

N = 320000
H = 128
G = 512
NC = 2
NS = 16
NW = NC * NS
ROWS_W = N // NW
CH = 80
NCHUNK = ROWS_W // CH
EPS = 1e-5
_SQRT2 = 1.4142135623730951


def _row0(wid, c):
    return wid * ROWS_W + c * CH


def _seg_sum_body(x_hbm, b3_hbm, sums_hbm, cnts_hbm,
                  xb0, xb1, xb2, xb3, idx_all, cnt_local, acc,
                  sx0, sx1, sx2, sx3, sc0, sc1, sc2, sc3):
    cid = lax.axis_index("core")
    sid = lax.axis_index("subcore")
    wid = cid * NS + sid
    xb = (xb0, xb1, xb2, xb3)
    sx = (sx0, sx1, sx2, sx3)
    ssc = (sc0, sc1, sc2, sc3)

    z16 = jnp.zeros((16,), jnp.float32)

    @pl.loop(0, 32)
    def _(i):
        for k in range(H // 16):
            xb0[i, pl.ds(k * 16, 16)] = z16

    @pl.loop(0, G // 16)
    def _(i):
        cnt_local[pl.ds(i * 16, 16)] = z16

    pltpu.sync_copy(xb0.at[pl.ds(0, 32)], acc.at[pl.ds(sid * 32, 32)])
    pltpu.sync_copy(b3_hbm.at[wid], idx_all)

    for b in range(2):
        pltpu.async_copy(x_hbm.at[pl.ds(_row0(wid, b), CH)], xb[b], sx[b])

    plsc.subcore_barrier()
    o16 = jnp.ones((16,), jnp.float32)

    def chunk_step(c, b, bi, issue_next):
        pltpu.make_async_copy(x_hbm.at[pl.ds(0, CH)], xb[b], sx[b]).wait()
        for j in range(CH // 16):
            idv = idx_all[c, pl.ds(j * 16, 16)]
            plsc.addupdate_scatter(cnt_local, [idv], o16)
        pltpu.async_copy(xb[b], acc.at[idx_all.at[c]], ssc[b], add=True)
        if issue_next:
            @pl.when(c + 2 < NCHUNK)
            def _():
                @pl.when(c >= 2)
                def _():
                    pltpu.make_async_copy(xb[bi], acc.at[idx_all.at[0]],
                                          ssc[bi]).wait()
                pltpu.async_copy(x_hbm.at[pl.ds(_row0(wid, c) + 2 * CH, CH)],
                                 xb[bi], sx[bi])

    @pl.loop(0, NCHUNK - 1, step=4)
    def _(cc):
        for b in range(4):
            chunk_step(cc + b, b, (b + 2) % 4, True)

    chunk_step(NCHUNK - 1, (NCHUNK - 1) % 4, (NCHUNK + 1) % 4, False)

    for b in range(4):
        pltpu.make_async_copy(xb[b], acc.at[idx_all.at[0]], ssc[b]).wait()

    pltpu.sync_copy(cnt_local, cnts_hbm.at[wid])
    plsc.subcore_barrier()
    pltpu.sync_copy(acc.at[pl.ds(sid * 32, 32)],
                    sums_hbm.at[cid, pl.ds(sid * 32, 32)])


def _mlp_body(sums_ref, cnts_ref, emb_ref, g_ref, b_ref, w1_ref, b1_ref,
              w2_ref, b2_ref, h_ref):
    sums = sums_ref[0] + sums_ref[1]
    cnt = jnp.sum(cnts_ref[...], axis=0)
    gx = sums / jnp.maximum(cnt, 1.0)[:, None]
    vn = emb_ref[...] + gx
    mu = jnp.mean(vn, axis=-1, keepdims=True)
    var = jnp.mean((vn - mu) * (vn - mu), axis=-1, keepdims=True)
    hn = (vn - mu) * lax.rsqrt(var + EPS) * g_ref[...] + b_ref[...]
    h1 = jnp.dot(hn, w1_ref[...], preferred_element_type=jnp.float32)
    h1 = h1 + b1_ref[...]
    h1 = 0.5 * h1 * (1.0 + lax.erf(h1 / _SQRT2))
    h2 = jnp.dot(h1, w2_ref[...], preferred_element_type=jnp.float32)
    h_ref[...] = h2 + b2_ref[...]


def _apply_body(x_hbm, b3_hbm, h_hbm, out_hbm,
                xb0, xb1, hb0, hb1, ob0, ob1, idx_all,
                sx0, sx1, sg0, sg1, so0, so1):
    cid = lax.axis_index("core")
    sid = lax.axis_index("subcore")
    wid = cid * NS + sid
    xb = (xb0, xb1)
    hb = (hb0, hb1)
    ob = (ob0, ob1)
    sx = (sx0, sx1)
    sg = (sg0, sg1)
    so = (so0, so1)

    pltpu.sync_copy(b3_hbm.at[wid], idx_all)
    for b in range(2):
        pltpu.async_copy(x_hbm.at[pl.ds(_row0(wid, b), CH)], xb[b], sx[b])
        pltpu.async_copy(h_hbm.at[idx_all.at[b]], hb[b], sg[b])

    def chunk_step(c, b, issue_next):
        pltpu.make_async_copy(x_hbm.at[pl.ds(0, CH)], xb[b], sx[b]).wait()
        pltpu.make_async_copy(h_hbm.at[idx_all.at[0]], hb[b], sg[b]).wait()

        @pl.when(c >= 2)
        def _():
            pltpu.make_async_copy(ob[b], out_hbm.at[pl.ds(0, CH)],
                                  so[b]).wait()

        @plsc.parallel_loop(0, CH, unroll=2)
        def _(i):
            for k in range(H // 16):
                s = pl.ds(k * 16, 16)
                ob[b][i, s] = xb[b][i, s] + hb[b][i, s]

        pltpu.async_copy(ob[b], out_hbm.at[pl.ds(_row0(wid, c), CH)], so[b])
        if issue_next:
            @pl.when(c + 2 < NCHUNK)
            def _():
                pltpu.async_copy(
                    x_hbm.at[pl.ds(_row0(wid, c) + 2 * CH, CH)], xb[b], sx[b])
                pltpu.async_copy(h_hbm.at[idx_all.at[c + 2]], hb[b], sg[b])

    @pl.loop(0, NCHUNK - 1, step=2)
    def _(cc):
        for b in range(2):
            chunk_step(cc + b, b, True)

    chunk_step(NCHUNK - 1, (NCHUNK - 1) % 2, False)
    for b in range(2):
        pltpu.make_async_copy(ob[b], out_hbm.at[pl.ds(0, CH)], so[b]).wait()


_sc_mesh = plsc.VectorSubcoreMesh(core_axis_name="core",
                                  subcore_axis_name="subcore")

_sc_cp = pltpu.CompilerParams()
if "needs_layout_passes" in pltpu.CompilerParams.__dataclass_fields__:
    _sc_cp = dataclasses.replace(_sc_cp, needs_layout_passes=False)

_seg_sum = pl.kernel(
    _seg_sum_body,
    out_type=(jax.ShapeDtypeStruct((NC, G, H), jnp.float32),
              jax.ShapeDtypeStruct((NW, G), jnp.float32)),
    mesh=_sc_mesh,
    compiler_params=_sc_cp,
    name="seg_sum_sc",
    scratch_types=[
        pltpu.VMEM((CH, H), jnp.float32),
        pltpu.VMEM((CH, H), jnp.float32),
        pltpu.VMEM((CH, H), jnp.float32),
        pltpu.VMEM((CH, H), jnp.float32),
        pltpu.VMEM((NCHUNK, CH), jnp.int32),
        pltpu.VMEM((G,), jnp.float32),
        pltpu.VMEM_SHARED((G, H), jnp.float32),
        pltpu.SemaphoreType.DMA,
        pltpu.SemaphoreType.DMA,
        pltpu.SemaphoreType.DMA,
        pltpu.SemaphoreType.DMA,
        pltpu.SemaphoreType.DMA,
        pltpu.SemaphoreType.DMA,
        pltpu.SemaphoreType.DMA,
        pltpu.SemaphoreType.DMA,
    ],
)

_apply = pl.kernel(
    _apply_body,
    out_type=jax.ShapeDtypeStruct((N, H), jnp.float32),
    mesh=_sc_mesh,
    compiler_params=_sc_cp,
    name="apply_sc",
    scratch_types=[
        pltpu.VMEM((CH, H), jnp.float32),
        pltpu.VMEM((CH, H), jnp.float32),
        pltpu.VMEM((CH, H), jnp.float32),
        pltpu.VMEM((CH, H), jnp.float32),
        pltpu.VMEM((CH, H), jnp.float32),
        pltpu.VMEM((CH, H), jnp.float32),
        pltpu.VMEM((NCHUNK, CH), jnp.int32),
        pltpu.SemaphoreType.DMA,
        pltpu.SemaphoreType.DMA,
        pltpu.SemaphoreType.DMA,
        pltpu.SemaphoreType.DMA,
        pltpu.SemaphoreType.DMA,
        pltpu.SemaphoreType.DMA,
    ],
)

_mlp = pl.pallas_call(
    _mlp_body,
    out_shape=jax.ShapeDtypeStruct((G, H), jnp.float32),
)


def kernel(x, batch, vn_embedding, ln_g, ln_b, W1, b1, W2, b2):
    batch3 = batch.astype(jnp.int32).reshape(NW, NCHUNK, CH)
    sums_p, cnts_p = _seg_sum(x, batch3)
    h = _mlp(sums_p, cnts_p, vn_embedding,
             ln_g.reshape(1, H), ln_b.reshape(1, H),
             W1, b1.reshape(1, H), W2, b2.reshape(1, H))
    x_out = _apply(x, batch3, h)
    return (x_out, h)


# speedup vs baseline: 5.4532x; 2.9041x over previous
import dataclasses

import jax
import jax.numpy as jnp
from jax import lax
from jax.experimental import pallas as pl
from jax.experimental.pallas import tpu as pltpu
from jax.experimental.pallas import tpu_sc as plsc

N = 320000
H = 128
G = 512
NC = 2
NS = 16
NW = NC * NS
ROWS_W = N // NW
CH = 80
NCHUNK = ROWS_W // CH
EPS = 1e-5
_SQRT2 = 1.4142135623730951


def _row0(wid, c):
    return wid * ROWS_W + c * CH


def _seg_sum_body(x_hbm, b3_hbm, sums_hbm, cnts_hbm,
                  xb0, xb1, xb2, xb3, idx_all, cnt_local, acc,
                  sx0, sx1, sx2, sx3, sc0, sc1, sc2, sc3):
    cid = lax.axis_index("core")
    sid = lax.axis_index("subcore")
    wid = cid * NS + sid
    xb = (xb0, xb1, xb2, xb3)
    sx = (sx0, sx1, sx2, sx3)
    ssc = (sc0, sc1, sc2, sc3)

    z16 = jnp.zeros((16,), jnp.float32)

    @pl.loop(0, 32)
    def _(i):
        for k in range(H // 16):
            xb0[i, pl.ds(k * 16, 16)] = z16

    @pl.loop(0, G // 16)
    def _(i):
        cnt_local[pl.ds(i * 16, 16)] = z16

    pltpu.sync_copy(xb0.at[pl.ds(0, 32)], acc.at[pl.ds(sid * 32, 32)])
    pltpu.sync_copy(b3_hbm.at[wid], idx_all)

    for b in range(2):
        pltpu.async_copy(x_hbm.at[pl.ds(_row0(wid, b), CH)], xb[b], sx[b])

    plsc.subcore_barrier()
    o16 = jnp.ones((16,), jnp.float32)

    def chunk_step(c, b, bi, issue_next):
        pltpu.make_async_copy(x_hbm.at[pl.ds(0, CH)], xb[b], sx[b]).wait()
        for j in range(CH // 16):
            idv = idx_all[c, pl.ds(j * 16, 16)]
            plsc.addupdate_scatter(cnt_local, [idv], o16)
        pltpu.async_copy(xb[b], acc.at[idx_all.at[c]], ssc[b], add=True)
        if issue_next:
            @pl.when(c + 2 < NCHUNK)
            def _():
                @pl.when(c >= 2)
                def _():
                    pltpu.make_async_copy(xb[bi], acc.at[idx_all.at[0]],
                                          ssc[bi]).wait()
                pltpu.async_copy(x_hbm.at[pl.ds(_row0(wid, c) + 2 * CH, CH)],
                                 xb[bi], sx[bi])

    @pl.loop(0, NCHUNK - 1, step=4)
    def _(cc):
        for b in range(4):
            chunk_step(cc + b, b, (b + 2) % 4, True)

    chunk_step(NCHUNK - 1, (NCHUNK - 1) % 4, (NCHUNK + 1) % 4, False)

    for b in range(4):
        pltpu.make_async_copy(xb[b], acc.at[idx_all.at[0]], ssc[b]).wait()

    pltpu.sync_copy(cnt_local, cnts_hbm.at[wid])
    plsc.subcore_barrier()
    pltpu.sync_copy(acc.at[pl.ds(sid * 32, 32)],
                    sums_hbm.at[cid, pl.ds(sid * 32, 32)])


def _mlp_body(sums_ref, cnts_ref, emb_ref, g_ref, b_ref, w1_ref, b1_ref,
              w2_ref, b2_ref, h_ref):
    sums = sums_ref[0] + sums_ref[1]
    cnt = jnp.sum(cnts_ref[...], axis=0)
    gx = sums / jnp.maximum(cnt, 1.0)[:, None]
    vn = emb_ref[...] + gx
    mu = jnp.mean(vn, axis=-1, keepdims=True)
    var = jnp.mean((vn - mu) * (vn - mu), axis=-1, keepdims=True)
    hn = (vn - mu) * lax.rsqrt(var + EPS) * g_ref[...] + b_ref[...]
    h1 = jnp.dot(hn, w1_ref[...], preferred_element_type=jnp.float32)
    h1 = h1 + b1_ref[...]
    h1 = 0.5 * h1 * (1.0 + lax.erf(h1 / _SQRT2))
    h2 = jnp.dot(h1, w2_ref[...], preferred_element_type=jnp.float32)
    h_ref[...] = h2 + b2_ref[...]


def _apply_body(x_hbm, b3_hbm, h_hbm, out_hbm,
                xb0, xb1, hb0, hb1, ob0, ob1, idx_all, hs,
                sx0, sx1, sg0, sg1, so0, so1):
    cid = lax.axis_index("core")
    sid = lax.axis_index("subcore")
    wid = cid * NS + sid
    xb = (xb0, xb1)
    hb = (hb0, hb1)
    ob = (ob0, ob1)
    sx = (sx0, sx1)
    sg = (sg0, sg1)
    so = (so0, so1)

    pltpu.sync_copy(b3_hbm.at[wid], idx_all)
    pltpu.sync_copy(h_hbm.at[pl.ds(sid * 32, 32)], hs.at[pl.ds(sid * 32, 32)])
    for b in range(2):
        pltpu.async_copy(x_hbm.at[pl.ds(_row0(wid, b), CH)], xb[b], sx[b])
    plsc.subcore_barrier()
    for b in range(2):
        pltpu.async_copy(hs.at[idx_all.at[b]], hb[b], sg[b])

    def chunk_step(c, b, issue_next):
        pltpu.make_async_copy(x_hbm.at[pl.ds(0, CH)], xb[b], sx[b]).wait()
        pltpu.make_async_copy(hs.at[idx_all.at[0]], hb[b], sg[b]).wait()

        @pl.when(c >= 2)
        def _():
            pltpu.make_async_copy(ob[b], out_hbm.at[pl.ds(0, CH)],
                                  so[b]).wait()

        @plsc.parallel_loop(0, CH, unroll=2)
        def _(i):
            for k in range(H // 16):
                s = pl.ds(k * 16, 16)
                ob[b][i, s] = xb[b][i, s] + hb[b][i, s]

        pltpu.async_copy(ob[b], out_hbm.at[pl.ds(_row0(wid, c), CH)], so[b])
        if issue_next:
            @pl.when(c + 2 < NCHUNK)
            def _():
                pltpu.async_copy(
                    x_hbm.at[pl.ds(_row0(wid, c) + 2 * CH, CH)], xb[b], sx[b])
                pltpu.async_copy(hs.at[idx_all.at[c + 2]], hb[b], sg[b])

    @pl.loop(0, NCHUNK - 1, step=2)
    def _(cc):
        for b in range(2):
            chunk_step(cc + b, b, True)

    chunk_step(NCHUNK - 1, (NCHUNK - 1) % 2, False)
    for b in range(2):
        pltpu.make_async_copy(ob[b], out_hbm.at[pl.ds(0, CH)], so[b]).wait()


_sc_mesh = plsc.VectorSubcoreMesh(core_axis_name="core",
                                  subcore_axis_name="subcore")

_sc_cp = pltpu.CompilerParams()
if "needs_layout_passes" in pltpu.CompilerParams.__dataclass_fields__:
    _sc_cp = dataclasses.replace(_sc_cp, needs_layout_passes=False)

_seg_sum = pl.kernel(
    _seg_sum_body,
    out_type=(jax.ShapeDtypeStruct((NC, G, H), jnp.float32),
              jax.ShapeDtypeStruct((NW, G), jnp.float32)),
    mesh=_sc_mesh,
    compiler_params=_sc_cp,
    name="seg_sum_sc",
    scratch_types=[
        pltpu.VMEM((CH, H), jnp.float32),
        pltpu.VMEM((CH, H), jnp.float32),
        pltpu.VMEM((CH, H), jnp.float32),
        pltpu.VMEM((CH, H), jnp.float32),
        pltpu.VMEM((NCHUNK, CH), jnp.int32),
        pltpu.VMEM((G,), jnp.float32),
        pltpu.VMEM_SHARED((G, H), jnp.float32),
        pltpu.SemaphoreType.DMA,
        pltpu.SemaphoreType.DMA,
        pltpu.SemaphoreType.DMA,
        pltpu.SemaphoreType.DMA,
        pltpu.SemaphoreType.DMA,
        pltpu.SemaphoreType.DMA,
        pltpu.SemaphoreType.DMA,
        pltpu.SemaphoreType.DMA,
    ],
)

_apply = pl.kernel(
    _apply_body,
    out_type=jax.ShapeDtypeStruct((N, H), jnp.float32),
    mesh=_sc_mesh,
    compiler_params=_sc_cp,
    name="apply_sc",
    scratch_types=[
        pltpu.VMEM((CH, H), jnp.float32),
        pltpu.VMEM((CH, H), jnp.float32),
        pltpu.VMEM((CH, H), jnp.float32),
        pltpu.VMEM((CH, H), jnp.float32),
        pltpu.VMEM((CH, H), jnp.float32),
        pltpu.VMEM((CH, H), jnp.float32),
        pltpu.VMEM((NCHUNK, CH), jnp.int32),
        pltpu.VMEM_SHARED((G, H), jnp.float32),
        pltpu.SemaphoreType.DMA,
        pltpu.SemaphoreType.DMA,
        pltpu.SemaphoreType.DMA,
        pltpu.SemaphoreType.DMA,
        pltpu.SemaphoreType.DMA,
        pltpu.SemaphoreType.DMA,
    ],
)

_mlp = pl.pallas_call(
    _mlp_body,
    out_shape=jax.ShapeDtypeStruct((G, H), jnp.float32),
)


def kernel(x, batch, vn_embedding, ln_g, ln_b, W1, b1, W2, b2):
    batch3 = batch.astype(jnp.int32).reshape(NW, NCHUNK, CH)
    sums_p, cnts_p = _seg_sum(x, batch3)
    h = _mlp(sums_p, cnts_p, vn_embedding,
             ln_g.reshape(1, H), ln_b.reshape(1, H),
             W1, b1.reshape(1, H), W2, b2.reshape(1, H))
    x_out = _apply(x, batch3, h)
    return (x_out, h)


# speedup vs baseline: 5.4637x; 1.0019x over previous
import dataclasses

import jax
import jax.numpy as jnp
from jax import lax
from jax.experimental import pallas as pl
from jax.experimental.pallas import tpu as pltpu
from jax.experimental.pallas import tpu_sc as plsc

N = 320000
H = 128
G = 512
NC = 2
NS = 16
NW = NC * NS
ROWS_W = N // NW
CH = 80
NCHUNK = ROWS_W // CH
EPS = 1e-5
_SQRT2 = 1.4142135623730951


def _row0(wid, c):
    return wid * ROWS_W + c * CH


NBA = 6


def _seg_sum_body(x_hbm, b3_hbm, sums_hbm, cnts_hbm,
                  xb0, xb1, xb2, xb3, xb4, xb5, idx_all, cnt_local, acc,
                  sx0, sx1, sx2, sx3, sx4, sx5,
                  sc0, sc1, sc2, sc3, sc4, sc5):
    cid = lax.axis_index("core")
    sid = lax.axis_index("subcore")
    wid = cid * NS + sid
    xb = (xb0, xb1, xb2, xb3, xb4, xb5)
    sx = (sx0, sx1, sx2, sx3, sx4, sx5)
    ssc = (sc0, sc1, sc2, sc3, sc4, sc5)

    z16 = jnp.zeros((16,), jnp.float32)

    @pl.loop(0, 32)
    def _(i):
        for k in range(H // 16):
            xb0[i, pl.ds(k * 16, 16)] = z16

    @pl.loop(0, G // 16)
    def _(i):
        cnt_local[pl.ds(i * 16, 16)] = z16

    pltpu.sync_copy(xb0.at[pl.ds(0, 32)], acc.at[pl.ds(sid * 32, 32)])
    pltpu.sync_copy(b3_hbm.at[wid], idx_all)

    for b in range(3):
        pltpu.async_copy(x_hbm.at[pl.ds(_row0(wid, b), CH)], xb[b], sx[b])

    plsc.subcore_barrier()
    o16 = jnp.ones((16,), jnp.float32)

    def chunk_step(c, b, bi, issue_next):
        pltpu.make_async_copy(x_hbm.at[pl.ds(0, CH)], xb[b], sx[b]).wait()
        for j in range(CH // 16):
            idv = idx_all[c, pl.ds(j * 16, 16)]
            plsc.addupdate_scatter(cnt_local, [idv], o16)
        pltpu.async_copy(xb[b], acc.at[idx_all.at[c]], ssc[b], add=True)
        if issue_next:
            @pl.when(c + 3 < NCHUNK)
            def _():
                @pl.when(c >= 3)
                def _():
                    pltpu.make_async_copy(xb[bi], acc.at[idx_all.at[0]],
                                          ssc[bi]).wait()
                pltpu.async_copy(x_hbm.at[pl.ds(_row0(wid, c) + 3 * CH, CH)],
                                 xb[bi], sx[bi])

    @pl.loop(0, NCHUNK - 5, step=NBA)
    def _(cc):
        for b in range(NBA):
            chunk_step(cc + b, b, (b + 3) % NBA, True)

    for c in range(NCHUNK - 5, NCHUNK):
        chunk_step(c, c % NBA, (c + 3) % NBA, c + 3 < NCHUNK)

    for b in range(NBA):
        pltpu.make_async_copy(xb[b], acc.at[idx_all.at[0]], ssc[b]).wait()

    pltpu.sync_copy(cnt_local, cnts_hbm.at[wid])
    plsc.subcore_barrier()
    pltpu.sync_copy(acc.at[pl.ds(sid * 32, 32)],
                    sums_hbm.at[cid, pl.ds(sid * 32, 32)])


def _mlp_body(sums_ref, cnts_ref, emb_ref, g_ref, b_ref, w1_ref, b1_ref,
              w2_ref, b2_ref, h_ref):
    sums = sums_ref[0] + sums_ref[1]
    cnt = jnp.sum(cnts_ref[...], axis=0)
    gx = sums / jnp.maximum(cnt, 1.0)[:, None]
    vn = emb_ref[...] + gx
    mu = jnp.mean(vn, axis=-1, keepdims=True)
    var = jnp.mean((vn - mu) * (vn - mu), axis=-1, keepdims=True)
    hn = (vn - mu) * lax.rsqrt(var + EPS) * g_ref[...] + b_ref[...]
    h1 = jnp.dot(hn, w1_ref[...], preferred_element_type=jnp.float32)
    h1 = h1 + b1_ref[...]
    h1 = 0.5 * h1 * (1.0 + lax.erf(h1 / _SQRT2))
    h2 = jnp.dot(h1, w2_ref[...], preferred_element_type=jnp.float32)
    h_ref[...] = h2 + b2_ref[...]


NBC = 3


def _apply_body(x_hbm, b3_hbm, h_hbm, out_hbm,
                xb0, xb1, xb2, hb0, hb1, hb2, ob0, ob1, ob2, idx_all, hs,
                sx0, sx1, sx2, sg0, sg1, sg2, so0, so1, so2):
    cid = lax.axis_index("core")
    sid = lax.axis_index("subcore")
    wid = cid * NS + sid
    xb = (xb0, xb1, xb2)
    hb = (hb0, hb1, hb2)
    ob = (ob0, ob1, ob2)
    sx = (sx0, sx1, sx2)
    sg = (sg0, sg1, sg2)
    so = (so0, so1, so2)

    pltpu.sync_copy(b3_hbm.at[wid], idx_all)
    pltpu.sync_copy(h_hbm.at[pl.ds(sid * 32, 32)], hs.at[pl.ds(sid * 32, 32)])
    for b in range(NBC):
        pltpu.async_copy(x_hbm.at[pl.ds(_row0(wid, b), CH)], xb[b], sx[b])
    plsc.subcore_barrier()
    for b in range(NBC):
        pltpu.async_copy(hs.at[idx_all.at[b]], hb[b], sg[b])

    def chunk_step(c, b, issue_next):
        pltpu.make_async_copy(x_hbm.at[pl.ds(0, CH)], xb[b], sx[b]).wait()
        pltpu.make_async_copy(hs.at[idx_all.at[0]], hb[b], sg[b]).wait()

        @pl.when(c >= NBC)
        def _():
            pltpu.make_async_copy(ob[b], out_hbm.at[pl.ds(0, CH)],
                                  so[b]).wait()

        @plsc.parallel_loop(0, CH, unroll=4)
        def _(i):
            for k in range(H // 16):
                s = pl.ds(k * 16, 16)
                ob[b][i, s] = xb[b][i, s] + hb[b][i, s]

        pltpu.async_copy(ob[b], out_hbm.at[pl.ds(_row0(wid, c), CH)], so[b])
        if issue_next:
            @pl.when(c + NBC < NCHUNK)
            def _():
                pltpu.async_copy(
                    x_hbm.at[pl.ds(_row0(wid, c) + NBC * CH, CH)],
                    xb[b], sx[b])
                pltpu.async_copy(hs.at[idx_all.at[c + NBC]], hb[b], sg[b])

    @pl.loop(0, NCHUNK - 2, step=NBC)
    def _(cc):
        for b in range(NBC):
            chunk_step(cc + b, b, True)

    for c in range(NCHUNK - 2, NCHUNK):
        chunk_step(c, c % NBC, False)

    for b in range(NBC):
        pltpu.make_async_copy(ob[b], out_hbm.at[pl.ds(0, CH)], so[b]).wait()


_sc_mesh = plsc.VectorSubcoreMesh(core_axis_name="core",
                                  subcore_axis_name="subcore")

_sc_cp = pltpu.CompilerParams()
if "needs_layout_passes" in pltpu.CompilerParams.__dataclass_fields__:
    _sc_cp = dataclasses.replace(_sc_cp, needs_layout_passes=False)

_seg_sum = pl.kernel(
    _seg_sum_body,
    out_type=(jax.ShapeDtypeStruct((NC, G, H), jnp.float32),
              jax.ShapeDtypeStruct((NW, G), jnp.float32)),
    mesh=_sc_mesh,
    compiler_params=_sc_cp,
    name="seg_sum_sc",
    scratch_types=(
        [pltpu.VMEM((CH, H), jnp.float32)] * NBA
        + [pltpu.VMEM((NCHUNK, CH), jnp.int32),
           pltpu.VMEM((G,), jnp.float32),
           pltpu.VMEM_SHARED((G, H), jnp.float32)]
        + [pltpu.SemaphoreType.DMA] * (2 * NBA)
    ),
)

_apply = pl.kernel(
    _apply_body,
    out_type=jax.ShapeDtypeStruct((N, H), jnp.float32),
    mesh=_sc_mesh,
    compiler_params=_sc_cp,
    name="apply_sc",
    scratch_types=(
        [pltpu.VMEM((CH, H), jnp.float32)] * (3 * NBC)
        + [pltpu.VMEM((NCHUNK, CH), jnp.int32),
           pltpu.VMEM_SHARED((G, H), jnp.float32)]
        + [pltpu.SemaphoreType.DMA] * (3 * NBC)
    ),
)

_mlp = pl.pallas_call(
    _mlp_body,
    out_shape=jax.ShapeDtypeStruct((G, H), jnp.float32),
)


def kernel(x, batch, vn_embedding, ln_g, ln_b, W1, b1, W2, b2):
    batch3 = batch.astype(jnp.int32).reshape(NW, NCHUNK, CH)
    sums_p, cnts_p = _seg_sum(x, batch3)
    h = _mlp(sums_p, cnts_p, vn_embedding,
             ln_g.reshape(1, H), ln_b.reshape(1, H),
             W1, b1.reshape(1, H), W2, b2.reshape(1, H))
    x_out = _apply(x, batch3, h)
    return (x_out, h)


# speedup vs baseline: 5.4750x; 1.0021x over previous
import dataclasses

import jax
import jax.numpy as jnp
from jax import lax
from jax.experimental import pallas as pl
from jax.experimental.pallas import tpu as pltpu
from jax.experimental.pallas import tpu_sc as plsc

N = 320000
H = 128
G = 512
NC = 2
NS = 16
NW = NC * NS
ROWS_W = N // NW
CH = 80
NCHUNK = ROWS_W // CH
EPS = 1e-5
_SQRT2 = 1.4142135623730951


def _row0(wid, c):
    return wid * ROWS_W + c * CH


NBA = 8


def _seg_sum_body(x_hbm, b3_hbm, sums_hbm, cnts_hbm,
                  xb0, xb1, xb2, xb3, xb4, xb5, xb6, xb7,
                  idx_all, cnt_local, acc,
                  sx0, sx1, sx2, sx3, sx4, sx5, sx6, sx7,
                  sc0, sc1, sc2, sc3, sc4, sc5, sc6, sc7):
    cid = lax.axis_index("core")
    sid = lax.axis_index("subcore")
    wid = cid * NS + sid
    xb = (xb0, xb1, xb2, xb3, xb4, xb5, xb6, xb7)
    sx = (sx0, sx1, sx2, sx3, sx4, sx5, sx6, sx7)
    ssc = (sc0, sc1, sc2, sc3, sc4, sc5, sc6, sc7)

    z16 = jnp.zeros((16,), jnp.float32)

    @pl.loop(0, 32)
    def _(i):
        for k in range(H // 16):
            xb0[i, pl.ds(k * 16, 16)] = z16

    @pl.loop(0, G // 16)
    def _(i):
        cnt_local[pl.ds(i * 16, 16)] = z16

    pltpu.sync_copy(xb0.at[pl.ds(0, 32)], acc.at[pl.ds(sid * 32, 32)])
    pltpu.sync_copy(b3_hbm.at[wid], idx_all)

    for b in range(3):
        pltpu.async_copy(x_hbm.at[pl.ds(_row0(wid, b), CH)], xb[b], sx[b])

    plsc.subcore_barrier()
    o16 = jnp.ones((16,), jnp.float32)

    def chunk_step(c, b, bi, issue_next):
        pltpu.make_async_copy(x_hbm.at[pl.ds(0, CH)], xb[b], sx[b]).wait()
        for j in range(CH // 16):
            idv = idx_all[c, pl.ds(j * 16, 16)]
            plsc.addupdate_scatter(cnt_local, [idv], o16)
        pltpu.async_copy(xb[b], acc.at[idx_all.at[c]], ssc[b], add=True)
        if issue_next:
            @pl.when(c + 3 < NCHUNK)
            def _():
                @pl.when(c >= 5)
                def _():
                    pltpu.make_async_copy(xb[bi], acc.at[idx_all.at[0]],
                                          ssc[bi]).wait()
                pltpu.async_copy(x_hbm.at[pl.ds(_row0(wid, c) + 3 * CH, CH)],
                                 xb[bi], sx[bi])

    @pl.loop(0, NCHUNK - 5, step=NBA)
    def _(cc):
        for b in range(NBA):
            chunk_step(cc + b, b, (b + 3) % NBA, True)

    for c in range(NCHUNK - 5, NCHUNK):
        chunk_step(c, c % NBA, (c + 3) % NBA, c + 3 < NCHUNK)

    for b in range(NBA):
        pltpu.make_async_copy(xb[b], acc.at[idx_all.at[0]], ssc[b]).wait()

    pltpu.sync_copy(cnt_local, cnts_hbm.at[wid])
    plsc.subcore_barrier()
    pltpu.sync_copy(acc.at[pl.ds(sid * 32, 32)],
                    sums_hbm.at[cid, pl.ds(sid * 32, 32)])


def _mlp_body(sums_ref, cnts_ref, emb_ref, g_ref, b_ref, w1_ref, b1_ref,
              w2_ref, b2_ref, h_ref):
    sums = sums_ref[0] + sums_ref[1]
    cnt = jnp.sum(cnts_ref[...], axis=0)
    gx = sums / jnp.maximum(cnt, 1.0)[:, None]
    vn = emb_ref[...] + gx
    mu = jnp.mean(vn, axis=-1, keepdims=True)
    var = jnp.mean((vn - mu) * (vn - mu), axis=-1, keepdims=True)
    hn = (vn - mu) * lax.rsqrt(var + EPS) * g_ref[...] + b_ref[...]
    h1 = jnp.dot(hn, w1_ref[...], preferred_element_type=jnp.float32)
    h1 = h1 + b1_ref[...]
    h1 = 0.5 * h1 * (1.0 + lax.erf(h1 / _SQRT2))
    h2 = jnp.dot(h1, w2_ref[...], preferred_element_type=jnp.float32)
    h_ref[...] = h2 + b2_ref[...]


NBC = 3


def _apply_body(x_hbm, b3_hbm, h_hbm, out_hbm,
                xb0, xb1, xb2, hb0, hb1, hb2, ob0, ob1, ob2, idx_all, hs,
                sx0, sx1, sx2, sg0, sg1, sg2, so0, so1, so2):
    cid = lax.axis_index("core")
    sid = lax.axis_index("subcore")
    wid = cid * NS + sid
    xb = (xb0, xb1, xb2)
    hb = (hb0, hb1, hb2)
    ob = (ob0, ob1, ob2)
    sx = (sx0, sx1, sx2)
    sg = (sg0, sg1, sg2)
    so = (so0, so1, so2)

    pltpu.sync_copy(b3_hbm.at[wid], idx_all)
    pltpu.sync_copy(h_hbm.at[pl.ds(sid * 32, 32)], hs.at[pl.ds(sid * 32, 32)])
    for b in range(NBC):
        pltpu.async_copy(x_hbm.at[pl.ds(_row0(wid, b), CH)], xb[b], sx[b])
    plsc.subcore_barrier()
    for b in range(NBC):
        pltpu.async_copy(hs.at[idx_all.at[b]], hb[b], sg[b])

    def chunk_step(c, b, issue_next):
        pltpu.make_async_copy(x_hbm.at[pl.ds(0, CH)], xb[b], sx[b]).wait()
        pltpu.make_async_copy(hs.at[idx_all.at[0]], hb[b], sg[b]).wait()

        @pl.when(c >= NBC)
        def _():
            pltpu.make_async_copy(ob[b], out_hbm.at[pl.ds(0, CH)],
                                  so[b]).wait()

        @plsc.parallel_loop(0, CH, unroll=4)
        def _(i):
            for k in range(H // 16):
                s = pl.ds(k * 16, 16)
                ob[b][i, s] = xb[b][i, s] + hb[b][i, s]

        pltpu.async_copy(ob[b], out_hbm.at[pl.ds(_row0(wid, c), CH)], so[b])
        if issue_next:
            @pl.when(c + NBC < NCHUNK)
            def _():
                pltpu.async_copy(
                    x_hbm.at[pl.ds(_row0(wid, c) + NBC * CH, CH)],
                    xb[b], sx[b])
                pltpu.async_copy(hs.at[idx_all.at[c + NBC]], hb[b], sg[b])

    @pl.loop(0, NCHUNK - 2, step=NBC)
    def _(cc):
        for b in range(NBC):
            chunk_step(cc + b, b, True)

    for c in range(NCHUNK - 2, NCHUNK):
        chunk_step(c, c % NBC, False)

    for b in range(NBC):
        pltpu.make_async_copy(ob[b], out_hbm.at[pl.ds(0, CH)], so[b]).wait()


_sc_mesh = plsc.VectorSubcoreMesh(core_axis_name="core",
                                  subcore_axis_name="subcore")

_sc_cp = pltpu.CompilerParams()
if "needs_layout_passes" in pltpu.CompilerParams.__dataclass_fields__:
    _sc_cp = dataclasses.replace(_sc_cp, needs_layout_passes=False)

_seg_sum = pl.kernel(
    _seg_sum_body,
    out_type=(jax.ShapeDtypeStruct((NC, G, H), jnp.float32),
              jax.ShapeDtypeStruct((NW, G), jnp.float32)),
    mesh=_sc_mesh,
    compiler_params=_sc_cp,
    name="seg_sum_sc",
    scratch_types=(
        [pltpu.VMEM((CH, H), jnp.float32)] * NBA
        + [pltpu.VMEM((NCHUNK, CH), jnp.int32),
           pltpu.VMEM((G,), jnp.float32),
           pltpu.VMEM_SHARED((G, H), jnp.float32)]
        + [pltpu.SemaphoreType.DMA] * (2 * NBA)
    ),
)

_apply = pl.kernel(
    _apply_body,
    out_type=jax.ShapeDtypeStruct((N, H), jnp.float32),
    mesh=_sc_mesh,
    compiler_params=_sc_cp,
    name="apply_sc",
    scratch_types=(
        [pltpu.VMEM((CH, H), jnp.float32)] * (3 * NBC)
        + [pltpu.VMEM((NCHUNK, CH), jnp.int32),
           pltpu.VMEM_SHARED((G, H), jnp.float32)]
        + [pltpu.SemaphoreType.DMA] * (3 * NBC)
    ),
)

_mlp = pl.pallas_call(
    _mlp_body,
    out_shape=jax.ShapeDtypeStruct((G, H), jnp.float32),
)


def kernel(x, batch, vn_embedding, ln_g, ln_b, W1, b1, W2, b2):
    batch3 = batch.astype(jnp.int32).reshape(NW, NCHUNK, CH)
    sums_p, cnts_p = _seg_sum(x, batch3)
    h = _mlp(sums_p, cnts_p, vn_embedding,
             ln_g.reshape(1, H), ln_b.reshape(1, H),
             W1, b1.reshape(1, H), W2, b2.reshape(1, H))
    x_out = _apply(x, batch3, h)
    return (x_out, h)
